# baseline (speedup 1.0000x reference)
import functools
import math

import jax
import jax.numpy as jnp
from jax.experimental import pallas as pl
from jax.experimental.pallas import tpu as pltpu

_H = 128
_NUM_GRAPHS = 16
_DEPTH = 3


def _rup(n, m=256):
    return ((n + m - 1) // m) * m


def _mm_kernel(a_ref, b_ref, o_ref, *, trans_b, diag_one_n, bm, bn):
    @pl.when(pl.program_id(2) == 0)
    def _init():
        o_ref[...] = jnp.zeros_like(o_ref)
    a = a_ref[...]
    b = b_ref[...]
    if trans_b:
        acc = jax.lax.dot_general(a, b, (((1,), (1,)), ((), ())),
                                  preferred_element_type=jnp.float32)
    else:
        acc = jnp.dot(a, b, preferred_element_type=jnp.float32)
    o_ref[...] += acc
    if diag_one_n is not None:
        @pl.when(pl.program_id(2) == pl.num_programs(2) - 1)
        def _diag():
            rows = pl.program_id(0) * bm + jax.lax.broadcasted_iota(
                jnp.int32, (bm, bn), 0)
            cols = pl.program_id(1) * bn + jax.lax.broadcasted_iota(
                jnp.int32, (bm, bn), 1)
            o_ref[...] = jnp.where((rows == cols) & (rows < diag_one_n),
                                   1.0, o_ref[...])


def _mm(a, b, trans_b=False, bm=256, bn=256, bk=256, diag_one_n=None):
    m, k = a.shape
    n = b.shape[0] if trans_b else b.shape[1]
    bm = min(bm, m)
    bn = min(bn, n)
    bk = min(bk, k)
    if trans_b:
        b_spec = pl.BlockSpec((bn, bk), lambda i, j, q: (j, q))
    else:
        b_spec = pl.BlockSpec((bk, bn), lambda i, j, q: (q, j))
    return pl.pallas_call(
        functools.partial(_mm_kernel, trans_b=trans_b, diag_one_n=diag_one_n,
                          bm=bm, bn=bn),
        grid=(m // bm, n // bn, k // bk),
        in_specs=[pl.BlockSpec((bm, bk), lambda i, j, q: (i, q)), b_spec],
        out_specs=pl.BlockSpec((bm, bn), lambda i, j, q: (i, j)),
        out_shape=jax.ShapeDtypeStruct((m, n), jnp.float32),
        compiler_params=pltpu.CompilerParams(
            dimension_semantics=("parallel", "parallel", "arbitrary")),
    )(a, b)


def _tr_kernel(a_ref, o_ref):
    o_ref[...] = a_ref[...].T


def _transpose(a, blk=256):
    m, n = a.shape
    blk = min(blk, m, n)
    return pl.pallas_call(
        _tr_kernel,
        grid=(m // blk, n // blk),
        in_specs=[pl.BlockSpec((blk, blk), lambda i, j: (i, j))],
        out_specs=pl.BlockSpec((blk, blk), lambda i, j: (j, i)),
        out_shape=jax.ShapeDtypeStruct((n, m), jnp.float32),
    )(a)


def _head_kernel(g_ref, lw_ref, lb_ref, bg_ref, bb_ref, ow_ref, ob_ref,
                 o_ref):
    inv = 1.0 / jnp.sqrt(1.0 + 1e-5)
    g = g_ref[...]
    for i in range(3):
        g = g * inv * bg_ref[i][None, :] + bb_ref[i][None, :]
        g = jnp.tanh(jnp.dot(g, lw_ref[i], preferred_element_type=jnp.float32)
                     + lb_ref[i][None, :])
    g = g * inv * bg_ref[3][None, :] + bb_ref[3][None, :]
    o_ref[...] = (jnp.dot(g, ow_ref[...], preferred_element_type=jnp.float32)
                  + ob_ref[0][None, :])


def _head(g0, lin_W, lin_b, bn_g, bn_b, out_W_pad, out_b_pad):
    return pl.pallas_call(
        _head_kernel,
        out_shape=jax.ShapeDtypeStruct((_NUM_GRAPHS, _H), jnp.float32),
    )(g0, lin_W, lin_b, bn_g, bn_b, out_W_pad, out_b_pad)


def _pad_rows(x, p):
    return jnp.pad(x, ((0, p - x.shape[0]), (0, 0)))


def _dense_dis(B, mask):
    deg = jnp.sum(B, axis=1) + mask
    return jnp.where(deg > 0.0, deg ** -0.5, 0.0)


def _gcn_dense(B, dis, mask, h_in, W, b, relu):
    y = dis[:, None] * _mm(h_in, W, bm=256, bn=128, bk=128)
    agg = _mm(B, y, bm=256, bn=128, bk=256) + y
    h = dis[:, None] * agg + b[None, :] * mask[:, None]
    return jnp.maximum(h, 0.0) if relu else h


def _score(h, w):
    return jnp.tanh((h @ w) / jnp.linalg.norm(w))


def kernel(x, edge_index, batch, down_W, down_b, pool_w, up_W, up_b,
           lin_W, lin_b, bn_g, bn_b, out_W, out_b):
    f32 = jnp.float32
    n0 = x.shape[0]
    L = [n0]
    for _ in range(_DEPTH):
        L.append(int(math.ceil(0.5 * L[-1])))
    P = [_rup(l) for l in L]

    src = edge_index[0]
    dst = edge_index[1]
    selfloop = src == dst

    upd = jnp.stack([jnp.ones_like(src, f32), selfloop.astype(f32)], axis=1)
    cnt = jnp.zeros((n0, 2), f32).at[dst].add(upd)
    indeg, selfc = cnt[:, 0], cnt[:, 1]
    dfix = jnp.where(selfc == 0.0, 2.0, 0.0)
    dis0 = (indeg + dfix) ** -0.5

    def gcn0(h_pad, W, b, relu):
        y = dis0[:, None] * _mm(h_pad, W, bm=256, bn=128, bk=128)[:n0]
        agg = jnp.zeros((n0, _H), f32).at[dst].add(y[src])
        h = dis0[:, None] * (agg + dfix[:, None] * y) + b[None, :]
        if relu:
            h = jnp.maximum(h, 0.0)
        return _pad_rows(h, P[0])

    x_pad = _pad_rows(x, P[0])
    h0 = gcn0(x_pad, down_W[0], down_b[0], relu=True)

    masks = [(jnp.arange(p) < l).astype(f32) for p, l in zip(P, L)]

    vals1, perm1 = jax.lax.top_k(_score(h0[:n0], pool_w[0]), L[1])
    inv1 = jnp.full((n0,), P[1], jnp.int32).at[perm1].set(
        jnp.arange(L[1], dtype=jnp.int32))
    keep = ~selfloop
    rd = jnp.where(keep, inv1[dst], P[1])
    rs = jnp.where(keep, inv1[src], P[1])
    ar1 = jnp.arange(L[1])
    bf16 = jnp.bfloat16
    one = jnp.ones((), bf16)
    Ar = jnp.zeros((P[1], P[0]), bf16).at[rd, src].add(one)
    Ar = Ar.at[ar1, perm1].add(one)
    Ac = jnp.zeros((P[0], P[1]), bf16).at[dst, rs].add(one)
    Ac = Ac.at[perm1, ar1].add(one)
    B1 = _mm(Ar, Ac, bm=512, bn=512, bk=1024, diag_one_n=L[1])
    dis1 = _dense_dis(B1, masks[1])
    h1_in = _pad_rows(h0[:n0][perm1] * vals1[:, None], P[1])
    h1 = _gcn_dense(B1, dis1, masks[1], h1_in, down_W[1], down_b[1], True)

    def next_level(B, T, h, lvl, pw):
        lp, lc = L[lvl - 1], L[lvl]
        pc = P[lvl]
        vals, perm = jax.lax.top_k(_score(h[:lp], pw), lc)
        permp = jnp.full((pc,), lp, jnp.int32).at[:lc].set(perm)
        Rr = B[permp, :].astype(jnp.bfloat16)
        RcT = T[permp, :].astype(jnp.bfloat16)
        Bn = _mm(Rr, RcT, trans_b=True, bm=256, bn=256, bk=512,
                 diag_one_n=lc)
        h_in = _pad_rows(h[:lp][perm] * vals[:, None], pc)
        return Bn, h_in, perm

    T1 = _transpose(B1)
    B2, h2_in, perm2 = next_level(B1, T1, h1, 2, pool_w[1])
    dis2 = _dense_dis(B2, masks[2])
    h2 = _gcn_dense(B2, dis2, masks[2], h2_in, down_W[2], down_b[2], True)

    T2 = _transpose(B2)
    B3, h3_in, perm3 = next_level(B2, T2, h2, 3, pool_w[2])
    dis3 = _dense_dis(B3, masks[3])
    h3 = _gcn_dense(B3, dis3, masks[3], h3_in, down_W[3], down_b[3], True)

    u = jnp.zeros((P[2], _H), f32).at[perm3].set(h3[:L[3]])
    h = _gcn_dense(B2, dis2, masks[2], h2 + u, up_W[0], up_b[0], True)

    u = jnp.zeros((P[1], _H), f32).at[perm2].set(h[:L[2]])
    h = _gcn_dense(B1, dis1, masks[1], h1 + u, up_W[1], up_b[1], True)

    u = jnp.zeros((n0, _H), f32).at[perm1].set(h[:L[1]])
    h = gcn0(_pad_rows(h0[:n0] + u, P[0]), up_W[2], up_b[2], relu=False)

    onehot = (batch[None, :] == jnp.arange(_NUM_GRAPHS)[:, None]).astype(f32)
    onehot = jnp.pad(onehot, ((0, 0), (0, P[0] - n0)))
    g0 = _mm(onehot, h, bm=16, bn=128, bk=256)
    ow = jnp.pad(out_W, ((0, 0), (0, _H - out_W.shape[1])))
    ob = jnp.pad(out_b, (0, _H - out_b.shape[0]))[None, :]
    out = _head(g0, lin_W, lin_b, bn_g, bn_b, ow, ob)
    return out[:, :out_W.shape[1]]


# speedup vs baseline: 1.0902x; 1.0902x over previous
import functools
import math

import jax
import jax.numpy as jnp
from jax import lax
from jax.experimental import pallas as pl
from jax.experimental.pallas import tpu as pltpu
from jax.experimental.pallas import tpu_sc as plsc

_H = 128
_NUM_GRAPHS = 16
_DEPTH = 3


def _rup(n, m=256):
    return ((n + m - 1) // m) * m


def _mm_kernel(a_ref, b_ref, o_ref, *, trans_b, diag_one_n, bm, bn):
    @pl.when(pl.program_id(2) == 0)
    def _init():
        o_ref[...] = jnp.zeros_like(o_ref)
    a = a_ref[...]
    b = b_ref[...]
    if trans_b:
        acc = jax.lax.dot_general(a, b, (((1,), (1,)), ((), ())),
                                  preferred_element_type=jnp.float32)
    else:
        acc = jnp.dot(a, b, preferred_element_type=jnp.float32)
    o_ref[...] += acc
    if diag_one_n is not None:
        @pl.when(pl.program_id(2) == pl.num_programs(2) - 1)
        def _diag():
            rows = pl.program_id(0) * bm + jax.lax.broadcasted_iota(
                jnp.int32, (bm, bn), 0)
            cols = pl.program_id(1) * bn + jax.lax.broadcasted_iota(
                jnp.int32, (bm, bn), 1)
            o_ref[...] = jnp.where((rows == cols) & (rows < diag_one_n),
                                   1.0, o_ref[...])


def _mm(a, b, trans_b=False, bm=256, bn=256, bk=256, diag_one_n=None):
    m, k = a.shape
    n = b.shape[0] if trans_b else b.shape[1]
    bm = min(bm, m)
    bn = min(bn, n)
    bk = min(bk, k)
    if trans_b:
        b_spec = pl.BlockSpec((bn, bk), lambda i, j, q: (j, q))
    else:
        b_spec = pl.BlockSpec((bk, bn), lambda i, j, q: (q, j))
    return pl.pallas_call(
        functools.partial(_mm_kernel, trans_b=trans_b, diag_one_n=diag_one_n,
                          bm=bm, bn=bn),
        grid=(m // bm, n // bn, k // bk),
        in_specs=[pl.BlockSpec((bm, bk), lambda i, j, q: (i, q)), b_spec],
        out_specs=pl.BlockSpec((bm, bn), lambda i, j, q: (i, j)),
        out_shape=jax.ShapeDtypeStruct((m, n), jnp.float32),
        compiler_params=pltpu.CompilerParams(
            dimension_semantics=("parallel", "parallel", "arbitrary")),
    )(a, b)


def _tr_kernel(a_ref, o_ref):
    o_ref[...] = a_ref[...].T


def _transpose(a, blk=256):
    m, n = a.shape
    blk = min(blk, m, n)
    return pl.pallas_call(
        _tr_kernel,
        grid=(m // blk, n // blk),
        in_specs=[pl.BlockSpec((blk, blk), lambda i, j: (i, j))],
        out_specs=pl.BlockSpec((blk, blk), lambda i, j: (j, i)),
        out_shape=jax.ShapeDtypeStruct((n, m), jnp.float32),
    )(a)


_NW = 32
_EC = 128


def _sc_agg_body(y_hbm, src_hbm, dst_hbm, zrows_hbm, out_hbm,
                 idx_s, idx_d, rows_v, acc, sem):
    c = lax.axis_index("c")
    s = lax.axis_index("s")
    wid = s * 2 + c
    nchunks = src_hbm.shape[1]
    p0 = acc.shape[0]
    stripe = p0 // 16
    pltpu.sync_copy(zrows_hbm, acc.at[pl.ds(s * stripe, stripe)])
    plsc.subcore_barrier()

    def chunk(i, carry):
        pltpu.sync_copy(src_hbm.at[wid, i], idx_s)
        pltpu.async_copy(y_hbm.at[idx_s], rows_v, sem).wait()
        pltpu.sync_copy(dst_hbm.at[wid, i], idx_d)
        pltpu.sync_copy(rows_v, acc.at[idx_d], add=True)
        return carry

    lax.fori_loop(0, nchunks, chunk, 0)
    plsc.subcore_barrier()
    pltpu.sync_copy(acc.at[pl.ds(s * stripe, stripe)],
                    out_hbm.at[c].at[pl.ds(s * stripe, stripe)])


def _sc_edge_agg(y, srcp, dstp, zrows):
    p0 = y.shape[0]
    kern = pl.kernel(
        _sc_agg_body,
        out_type=jax.ShapeDtypeStruct((2, p0, _H), jnp.float32),
        mesh=plsc.VectorSubcoreMesh(core_axis_name="c", subcore_axis_name="s"),
        scratch_types=[
            pltpu.VMEM((_EC,), jnp.int32),
            pltpu.VMEM((_EC,), jnp.int32),
            pltpu.VMEM((_EC, _H), jnp.float32),
            pltpu.VMEM_SHARED((p0, _H), jnp.float32),
            pltpu.SemaphoreType.DMA,
        ],
    )
    return kern(y, srcp, dstp, zrows)


def _head_kernel(g_ref, lw_ref, lb_ref, bg_ref, bb_ref, ow_ref, ob_ref,
                 o_ref):
    inv = 1.0 / jnp.sqrt(1.0 + 1e-5)
    g = g_ref[...]
    for i in range(3):
        g = g * inv * bg_ref[i][None, :] + bb_ref[i][None, :]
        g = jnp.tanh(jnp.dot(g, lw_ref[i], preferred_element_type=jnp.float32)
                     + lb_ref[i][None, :])
    g = g * inv * bg_ref[3][None, :] + bb_ref[3][None, :]
    o_ref[...] = (jnp.dot(g, ow_ref[...], preferred_element_type=jnp.float32)
                  + ob_ref[0][None, :])


def _head(g0, lin_W, lin_b, bn_g, bn_b, out_W_pad, out_b_pad):
    return pl.pallas_call(
        _head_kernel,
        out_shape=jax.ShapeDtypeStruct((_NUM_GRAPHS, _H), jnp.float32),
    )(g0, lin_W, lin_b, bn_g, bn_b, out_W_pad, out_b_pad)


def _pad_rows(x, p):
    return jnp.pad(x, ((0, p - x.shape[0]), (0, 0)))


def _dense_dis(B, mask):
    deg = jnp.sum(B, axis=1) + mask
    return jnp.where(deg > 0.0, deg ** -0.5, 0.0)


def _gcn_dense(B, dis, mask, h_in, W, b, relu):
    y = dis[:, None] * _mm(h_in, W, bm=256, bn=128, bk=128)
    agg = _mm(B, y, bm=256, bn=128, bk=256) + y
    h = dis[:, None] * agg + b[None, :] * mask[:, None]
    return jnp.maximum(h, 0.0) if relu else h


def _score(h, w):
    return jnp.tanh((h @ w) / jnp.linalg.norm(w))


def kernel(x, edge_index, batch, down_W, down_b, pool_w, up_W, up_b,
           lin_W, lin_b, bn_g, bn_b, out_W, out_b):
    f32 = jnp.float32
    n0 = x.shape[0]
    L = [n0]
    for _ in range(_DEPTH):
        L.append(int(math.ceil(0.5 * L[-1])))
    P = [_rup(l) for l in L]

    src = edge_index[0]
    dst = edge_index[1]
    selfloop = src == dst

    indeg = jnp.zeros((n0,), f32).at[dst].add(1.0)
    selfc = jnp.zeros((n0,), f32).at[dst].add(selfloop.astype(f32))
    dfix = jnp.where(selfc == 0.0, 2.0, 0.0)
    dis0 = (indeg + dfix) ** -0.5
    dis0p = jnp.pad(dis0, (0, P[0] - n0))

    ne = edge_index.shape[1]
    npad = _NW * _EC * int(math.ceil(ne / (_NW * _EC)))
    srcp = jnp.pad(src, (0, npad - ne)).reshape(_NW, -1, _EC).astype(jnp.int32)
    dstp = jnp.pad(dst, (0, npad - ne), constant_values=n0)
    dstp = dstp.reshape(_NW, -1, _EC).astype(jnp.int32)
    zrows = jnp.zeros((P[0] // 16, _H), f32)

    def gcn0(h_pad, W, b, relu):
        y = dis0p[:, None] * _mm(h_pad, W, bm=256, bn=128, bk=128)
        parts = _sc_edge_agg(y, srcp, dstp, zrows)
        agg = parts[0, :n0] + parts[1, :n0]
        yl = y[:n0]
        h = dis0[:, None] * (agg + dfix[:, None] * yl) + b[None, :]
        if relu:
            h = jnp.maximum(h, 0.0)
        return _pad_rows(h, P[0])

    x_pad = _pad_rows(x, P[0])
    h0 = gcn0(x_pad, down_W[0], down_b[0], relu=True)

    masks = [(jnp.arange(p) < l).astype(f32) for p, l in zip(P, L)]

    vals1, perm1 = jax.lax.top_k(_score(h0[:n0], pool_w[0]), L[1])
    inv1 = jnp.full((n0,), P[1], jnp.int32).at[perm1].set(
        jnp.arange(L[1], dtype=jnp.int32))
    keep = ~selfloop
    rd = jnp.where(keep, inv1[dst], P[1])
    rs = jnp.where(keep, inv1[src], P[1])
    ar1 = jnp.arange(L[1])
    bf16 = jnp.bfloat16
    one = jnp.ones((), bf16)
    Ar = jnp.zeros((P[1], P[0]), bf16).at[rd, src].add(one)
    Ar = Ar.at[ar1, perm1].add(one)
    Ac = jnp.zeros((P[0], P[1]), bf16).at[dst, rs].add(one)
    Ac = Ac.at[perm1, ar1].add(one)
    B1 = _mm(Ar, Ac, bm=512, bn=512, bk=1024, diag_one_n=L[1])
    dis1 = _dense_dis(B1, masks[1])
    h1_in = _pad_rows(h0[:n0][perm1] * vals1[:, None], P[1])
    h1 = _gcn_dense(B1, dis1, masks[1], h1_in, down_W[1], down_b[1], True)

    def next_level(B, T, h, lvl, pw):
        lp, lc = L[lvl - 1], L[lvl]
        pc = P[lvl]
        vals, perm = jax.lax.top_k(_score(h[:lp], pw), lc)
        permp = jnp.full((pc,), lp, jnp.int32).at[:lc].set(perm)
        Rr = B[permp, :].astype(jnp.bfloat16)
        RcT = T[permp, :].astype(jnp.bfloat16)
        Bn = _mm(Rr, RcT, trans_b=True, bm=256, bn=256, bk=512,
                 diag_one_n=lc)
        h_in = _pad_rows(h[:lp][perm] * vals[:, None], pc)
        return Bn, h_in, perm

    T1 = _transpose(B1)
    B2, h2_in, perm2 = next_level(B1, T1, h1, 2, pool_w[1])
    dis2 = _dense_dis(B2, masks[2])
    h2 = _gcn_dense(B2, dis2, masks[2], h2_in, down_W[2], down_b[2], True)

    T2 = _transpose(B2)
    B3, h3_in, perm3 = next_level(B2, T2, h2, 3, pool_w[2])
    dis3 = _dense_dis(B3, masks[3])
    h3 = _gcn_dense(B3, dis3, masks[3], h3_in, down_W[3], down_b[3], True)

    u = jnp.zeros((P[2], _H), f32).at[perm3].set(h3[:L[3]])
    h = _gcn_dense(B2, dis2, masks[2], h2 + u, up_W[0], up_b[0], True)

    u = jnp.zeros((P[1], _H), f32).at[perm2].set(h[:L[2]])
    h = _gcn_dense(B1, dis1, masks[1], h1 + u, up_W[1], up_b[1], True)

    u = jnp.zeros((n0, _H), f32).at[perm1].set(h[:L[1]])
    h = gcn0(_pad_rows(h0[:n0] + u, P[0]), up_W[2], up_b[2], relu=False)

    onehot = (batch[None, :] == jnp.arange(_NUM_GRAPHS)[:, None]).astype(f32)
    onehot = jnp.pad(onehot, ((0, 0), (0, P[0] - n0)))
    g0 = _mm(onehot, h, bm=16, bn=128, bk=256)
    ow = jnp.pad(out_W, ((0, 0), (0, _H - out_W.shape[1])))
    ob = jnp.pad(out_b, (0, _H - out_b.shape[0]))[None, :]
    out = _head(g0, lin_W, lin_b, bn_g, bn_b, ow, ob)
    return out[:, :out_W.shape[1]]
